# baseline (speedup 1.0000x reference)
import functools

import jax
import jax.numpy as jnp
from jax import lax
from jax.experimental import pallas as pl
from jax.experimental.pallas import tpu as pltpu
from jax.experimental.pallas import tpu_sc as plsc

VOCAB = 100000
EMBED = 64
MAX_POS = 200
BATCH = 4096
SEQ = 200

NC = 2
NS = 16
NW = NC * NS
LANES = 16

BGRP = BATCH // NW
NGRP = EMBED // LANES
NSEG = BGRP // LANES


def _emb_body(table_hbm, pos_hbm, ids_hbm, out_hbm, pos_v, idx_v, rows_v, trans_v, sem):
    cid = lax.axis_index("c")
    sid = lax.axis_index("s")
    wid = sid * NC + cid
    b0 = wid * BGRP

    pltpu.sync_copy(pos_hbm, pos_v)

    iotas = [lax.iota(jnp.int32, LANES) + (LANES * g) for g in range(NSEG)]

    def block_body(l, carry):
        pltpu.sync_copy(ids_hbm.at[l, pl.ds(b0, BGRP)], idx_v)
        pltpu.async_copy(table_hbm.at[idx_v], rows_v, sem).wait()

        for j in range(NGRP):
            pv16 = pos_v[l, pl.ds(LANES * j, LANES)]

            def k_body(k, acc, pv16=pv16, j=j):
                e = LANES * j + k
                kvec = jnp.full((LANES,), k, jnp.int32)
                pe = pv16.at[kvec].get(mode="promise_in_bounds")
                col = jnp.full((LANES,), e, jnp.int32)
                for g in range(NSEG):
                    v = plsc.load_gather(rows_v, [iotas[g], col])
                    trans_v[e, pl.ds(LANES * g, LANES)] = v + pe
                return acc

            lax.fori_loop(0, LANES, k_body, 0)
        pltpu.sync_copy(trans_v, out_hbm.at[l, :, pl.ds(b0, BGRP)])
        return carry

    lax.fori_loop(0, MAX_POS, block_body, 0)


@jax.jit
def _emb(table, pos2d, ids_t):
    mesh = plsc.VectorSubcoreMesh(core_axis_name="c", subcore_axis_name="s")
    return pl.kernel(
        _emb_body,
        out_type=jax.ShapeDtypeStruct((MAX_POS, EMBED, BATCH), jnp.float32),
        mesh=mesh,
        scratch_types=[
            pltpu.VMEM((MAX_POS, EMBED), jnp.float32),
            pltpu.VMEM((BGRP,), jnp.int32),
            pltpu.VMEM((BGRP, EMBED), jnp.float32),
            pltpu.VMEM((EMBED, BGRP), jnp.float32),
            pltpu.SemaphoreType.DMA,
        ],
        compiler_params=pltpu.CompilerParams(use_tc_tiling_on_sc=False, needs_layout_passes=False),
    )(table, pos2d, ids_t)


def kernel(embedding_table, position_embeds, input_ids):
    ids_t = jnp.transpose(input_ids.astype(jnp.int32))
    pos2d = position_embeds.reshape(MAX_POS, EMBED)
    outp = _emb(embedding_table, pos2d, ids_t)
    return jnp.transpose(outp, (2, 0, 1))
